# baseline (speedup 1.0000x reference)
import jax
import jax.numpy as jnp
from jax import lax
from jax.experimental import pallas as pl
from jax.experimental.pallas import tpu as pltpu
from jax.experimental.pallas import tpu_sc as plsc

L = 16

OFF_CARD = 1001
OFF_POTION = 1033
OFF_PATH = 1073
OFF_LINKS = 1163
OFF_BOSS = 1283
OFF_SCAL = 1299
OUT_LEN = 1302
OUT_PAD = 1312

N_DECK = 100
DECK_PAD = 112
COLL_PAD = 1008
TAIL_PAD = 128
BOSS_ID_POS = 123


def _body(coll_h, deck_h, potion_h, path_h, tail_h, pot_emb_h, node_emb_h,
          boss_emb_h, card_emb_h, out_h,
          coll_v, deck_v, potion_v, path_v, tail_v, pot_emb_v, node_emb_v,
          boss_emb_v, cards_v, out_v, sem, gsem):
    @pl.when(jnp.logical_and(lax.axis_index("c") == 0, lax.axis_index("s") == 0))
    def _():
        copies = [
            pltpu.async_copy(deck_h, deck_v, sem),
            pltpu.async_copy(coll_h, coll_v, sem),
            pltpu.async_copy(potion_h, potion_v, sem),
            pltpu.async_copy(path_h, path_v, sem),
            pltpu.async_copy(tail_h, tail_v, sem),
            pltpu.async_copy(pot_emb_h, pot_emb_v, sem),
            pltpu.async_copy(node_emb_h, node_emb_v, sem),
            pltpu.async_copy(boss_emb_h, boss_emb_v, sem),
        ]
        for c in copies:
            c.wait()

        gather = pltpu.async_copy(card_emb_h.at[deck_v], cards_v, gsem)

        lanes = lax.iota(jnp.int32, L)

        for i in range(62):
            out_v[pl.ds(i * L, L)] = coll_v[pl.ds(i * L, L)].astype(jnp.float32)
        t = 62 * L + lanes
        plsc.store_scatter(out_v, [t],
                           coll_v[pl.ds(62 * L, L)].astype(jnp.float32),
                           mask=t < OFF_CARD)

        for c in range(3):
            t = c * L + lanes
            row = plsc.load_gather(potion_v, [t // 8])
            val = plsc.load_gather(pot_emb_v, [row * 8 + t % 8])
            plsc.store_scatter(out_v, [OFF_POTION + t], val, mask=t < 40)

        for c in range(6):
            t = c * L + lanes
            row = plsc.load_gather(path_v, [t // 6])
            val = plsc.load_gather(node_emb_v, [row * 6 + t % 6])
            plsc.store_scatter(out_v, [OFF_PATH + t], val, mask=t < 90)

        for c in range(8):
            t = c * L + lanes
            val = tail_v[pl.ds(c * L, L)].astype(jnp.float32)
            idx = jnp.where(t < 120, OFF_LINKS + t, OFF_SCAL + (t - 120))
            plsc.store_scatter(out_v, [idx], val, mask=t < 123)

        boss = plsc.load_gather(tail_v, [jnp.full((L,), BOSS_ID_POS, jnp.int32)])
        bval = plsc.load_gather(boss_emb_v, [boss * L + lanes])
        plsc.store_scatter(out_v, [OFF_BOSS + lanes], bval)

        gather.wait()
        acc0 = [None] * 4
        acc1 = [None] * 4
        for i in range(4):
            acc0[i] = cards_v[i, pl.ds(0, L)]
            acc1[i] = cards_v[i, pl.ds(L, L)]
        for i in range(4, N_DECK):
            k = i % 4
            acc0[k] = acc0[k] + cards_v[i, pl.ds(0, L)]
            acc1[k] = acc1[k] + cards_v[i, pl.ds(L, L)]
        scale = jnp.float32(1.0 / N_DECK)
        a0 = (acc0[0] + acc0[1]) + (acc0[2] + acc0[3])
        a1 = (acc1[0] + acc1[1]) + (acc1[2] + acc1[3])
        plsc.store_scatter(out_v, [OFF_CARD + lanes], a0 * scale)
        plsc.store_scatter(out_v, [OFF_CARD + L + lanes], a1 * scale)

        pltpu.sync_copy(out_v, out_h)


_encode = pl.kernel(
    _body,
    out_type=jax.ShapeDtypeStruct((OUT_PAD,), jnp.float32),
    mesh=plsc.VectorSubcoreMesh(core_axis_name="c", subcore_axis_name="s",
                                num_cores=1, num_subcores=1),
    compiler_params=pltpu.CompilerParams(needs_layout_passes=False,
                                         use_tc_tiling_on_sc=False,
                                         skip_device_barrier=True),
    scratch_types=[
        pltpu.VMEM((COLL_PAD,), jnp.int32),
        pltpu.VMEM((DECK_PAD,), jnp.int32),
        pltpu.VMEM((L,), jnp.int32),
        pltpu.VMEM((L,), jnp.int32),
        pltpu.VMEM((TAIL_PAD,), jnp.int32),
        pltpu.VMEM((416,), jnp.float32),
        pltpu.VMEM((64,), jnp.float32),
        pltpu.VMEM((320,), jnp.float32),
        pltpu.VMEM((DECK_PAD, 32), jnp.float32),
        pltpu.VMEM((OUT_PAD,), jnp.float32),
        pltpu.SemaphoreType.DMA,
        pltpu.SemaphoreType.DMA,
    ],
)


def kernel(collection, card_deck, potion, path_nodes, current_links,
           next_links, boss_id, current_health, max_health, current_floor,
           card_embed, potion_embed, node_embed, boss_embed):
    i32 = jnp.int32
    coll_p = jnp.pad(collection.astype(i32), (0, COLL_PAD - 1001))
    deck_p = jnp.pad(card_deck.astype(i32), (0, DECK_PAD - N_DECK))
    potion_p = jnp.pad(potion.astype(i32), (0, L - 5))
    path_p = jnp.pad(path_nodes.astype(i32), (0, L - 15))
    tail = jnp.concatenate([
        current_links.astype(i32).reshape(-1),
        next_links.astype(i32).reshape(-1),
        jnp.stack([jnp.asarray(current_health, i32),
                   jnp.asarray(max_health, i32),
                   jnp.asarray(current_floor, i32),
                   jnp.asarray(boss_id, i32)]),
    ])
    tail_p = jnp.pad(tail, (0, TAIL_PAD - 124))
    pot_emb_p = jnp.pad(potion_embed.reshape(-1), (0, 416 - 408))
    node_emb_p = jnp.pad(node_embed.reshape(-1), (0, 64 - 60))
    boss_emb_f = boss_embed.reshape(-1)
    out = _encode(coll_p, deck_p, potion_p, path_p, tail_p, pot_emb_p,
                  node_emb_p, boss_emb_f, card_embed)
    return out[:OUT_LEN]


# speedup vs baseline: 1.0760x; 1.0760x over previous
import jax
import jax.numpy as jnp
from jax import lax
from jax.experimental import pallas as pl
from jax.experimental.pallas import tpu as pltpu
from jax.experimental.pallas import tpu_sc as plsc

L = 16

OFF_CARD = 1001
OFF_POTION = 1033
OFF_PATH = 1073
OFF_LINKS = 1163
OFF_BOSS = 1283
OFF_SCAL = 1299
OUT_LEN = 1302
OUT_PAD = 1312

N_DECK = 100
DECK_PAD = 112

C_COLL = 0
C_POTION = 1008
C_PATH = 1024
C_TAIL = 1040
C_POT_EMB = 1168
C_NODE_EMB = 1584
C_BOSS_EMB = 1648
C_TOTAL = 1968
BOSS_ID_POS = C_TAIL + 123


def _body(comb_h, deck_h, card_emb_h, out_h,
          comb_v, deck_v, cards_v, out_v, sem, gsem):
    @pl.when(jnp.logical_and(lax.axis_index("c") == 0, lax.axis_index("s") == 0))
    def _():
        d1 = pltpu.async_copy(deck_h, deck_v, sem)
        d2 = pltpu.async_copy(comb_h, comb_v, sem)
        d1.wait()
        gather = pltpu.async_copy(card_emb_h.at[deck_v], cards_v, gsem)
        d2.wait()

        lanes = lax.iota(jnp.int32, L)
        f32 = jnp.float32

        for i in range(62):
            out_v[pl.ds(i * L, L)] = comb_v[pl.ds(C_COLL + i * L, L)].astype(f32)
        t = 62 * L + lanes
        plsc.store_scatter(out_v, [t],
                           comb_v[pl.ds(C_COLL + 62 * L, L)].astype(f32),
                           mask=t < OFF_CARD)

        for c in range(3):
            t = c * L + lanes
            row = plsc.load_gather(comb_v, [C_POTION + t // 8])
            val = plsc.load_gather(comb_v, [C_POT_EMB + row * 8 + t % 8])
            plsc.store_scatter(out_v, [OFF_POTION + t],
                               plsc.bitcast(val, f32), mask=t < 40)

        for c in range(6):
            t = c * L + lanes
            row = plsc.load_gather(comb_v, [C_PATH + t // 6])
            val = plsc.load_gather(comb_v, [C_NODE_EMB + row * 6 + t % 6])
            plsc.store_scatter(out_v, [OFF_PATH + t],
                               plsc.bitcast(val, f32), mask=t < 90)

        for c in range(8):
            t = c * L + lanes
            val = comb_v[pl.ds(C_TAIL + c * L, L)].astype(f32)
            idx = jnp.where(t < 120, OFF_LINKS + t, OFF_SCAL + (t - 120))
            plsc.store_scatter(out_v, [idx], val, mask=t < 123)

        boss = plsc.load_gather(comb_v, [jnp.full((L,), BOSS_ID_POS, jnp.int32)])
        bval = plsc.load_gather(comb_v, [C_BOSS_EMB + boss * L + lanes])
        plsc.store_scatter(out_v, [OFF_BOSS + lanes], plsc.bitcast(bval, f32))

        gather.wait()
        acc0 = [None] * 4
        acc1 = [None] * 4
        for i in range(4):
            acc0[i] = cards_v[i, pl.ds(0, L)]
            acc1[i] = cards_v[i, pl.ds(L, L)]
        for i in range(4, N_DECK):
            k = i % 4
            acc0[k] = acc0[k] + cards_v[i, pl.ds(0, L)]
            acc1[k] = acc1[k] + cards_v[i, pl.ds(L, L)]
        scale = f32(1.0 / N_DECK)
        a0 = (acc0[0] + acc0[1]) + (acc0[2] + acc0[3])
        a1 = (acc1[0] + acc1[1]) + (acc1[2] + acc1[3])
        plsc.store_scatter(out_v, [OFF_CARD + lanes], a0 * scale)
        plsc.store_scatter(out_v, [OFF_CARD + L + lanes], a1 * scale)

        pltpu.sync_copy(out_v, out_h)


_encode = pl.kernel(
    _body,
    out_type=jax.ShapeDtypeStruct((OUT_PAD,), jnp.float32),
    mesh=plsc.VectorSubcoreMesh(core_axis_name="c", subcore_axis_name="s",
                                num_cores=1, num_subcores=1),
    compiler_params=pltpu.CompilerParams(needs_layout_passes=False,
                                         use_tc_tiling_on_sc=False,
                                         skip_device_barrier=True),
    scratch_types=[
        pltpu.VMEM((C_TOTAL,), jnp.int32),
        pltpu.VMEM((DECK_PAD,), jnp.int32),
        pltpu.VMEM((DECK_PAD, 32), jnp.float32),
        pltpu.VMEM((OUT_PAD,), jnp.float32),
        pltpu.SemaphoreType.DMA,
        pltpu.SemaphoreType.DMA,
    ],
)


def kernel(collection, card_deck, potion, path_nodes, current_links,
           next_links, boss_id, current_health, max_health, current_floor,
           card_embed, potion_embed, node_embed, boss_embed):
    i32 = jnp.int32
    bits = lambda x: jax.lax.bitcast_convert_type(x.reshape(-1), i32)
    z = jnp.zeros((), i32)
    comb = jnp.concatenate([
        collection.astype(i32), jnp.zeros((7,), i32),
        potion.astype(i32), jnp.zeros((11,), i32),
        path_nodes.astype(i32), jnp.zeros((1,), i32),
        current_links.astype(i32).reshape(-1),
        next_links.astype(i32).reshape(-1),
        jnp.stack([jnp.asarray(current_health, i32),
                   jnp.asarray(max_health, i32),
                   jnp.asarray(current_floor, i32),
                   jnp.asarray(boss_id, i32)]),
        jnp.zeros((4,), i32),
        bits(potion_embed), jnp.zeros((8,), i32),
        bits(node_embed), jnp.zeros((4,), i32),
        bits(boss_embed),
    ])
    deck_p = jnp.pad(card_deck.astype(i32), (0, DECK_PAD - N_DECK))
    out = _encode(comb, deck_p, card_embed)
    return out[:OUT_LEN]


# speedup vs baseline: 1.0799x; 1.0036x over previous
import jax
import jax.numpy as jnp
from jax import lax
from jax.experimental import pallas as pl
from jax.experimental.pallas import tpu as pltpu
from jax.experimental.pallas import tpu_sc as plsc

L = 16

OFF_CARD = 1001
OFF_POTION = 1033
OFF_PATH = 1073
OFF_LINKS = 1163
OFF_BOSS = 1283
OFF_SCAL = 1299
OUT_LEN = 1302
OUT_PAD = 1312

N_DECK = 100
DECK_PAD = 112

C_COLL = 0
C_POTION = 1008
C_PATH = 1024
C_TAIL = 1040
C_POT_EMB = 1168
C_NODE_EMB = 1584
C_BOSS_EMB = 1648
C_TOTAL = 1968
BOSS_ID_POS = C_TAIL + 123


def _body(comb_h, deck_h, card_emb_h, out_h,
          comb_v, deck_v, cards_v, out_v, sem, gsem):
    @pl.when(jnp.logical_and(lax.axis_index("c") == 0, lax.axis_index("s") == 0))
    def _():
        d1 = pltpu.async_copy(deck_h, deck_v, sem)
        d2 = pltpu.async_copy(comb_h, comb_v, sem)
        d1.wait()
        gather = pltpu.async_copy(card_emb_h.at[deck_v], cards_v, gsem)
        d2.wait()

        lanes = lax.iota(jnp.int32, L)
        f32 = jnp.float32

        for i in range(62):
            out_v[pl.ds(i * L, L)] = comb_v[pl.ds(C_COLL + i * L, L)].astype(f32)
        t = 62 * L + lanes
        plsc.store_scatter(out_v, [t],
                           comb_v[pl.ds(C_COLL + 62 * L, L)].astype(f32),
                           mask=t < OFF_CARD)

        out1 = pltpu.async_copy(out_v.at[pl.ds(0, 960)],
                                out_h.at[pl.ds(0, 960)], sem)

        for c in range(3):
            t = c * L + lanes
            row = plsc.load_gather(comb_v, [C_POTION + t // 8])
            val = plsc.load_gather(comb_v, [C_POT_EMB + row * 8 + t % 8])
            plsc.store_scatter(out_v, [OFF_POTION + t],
                               plsc.bitcast(val, f32), mask=t < 40)

        for c in range(6):
            t = c * L + lanes
            row = plsc.load_gather(comb_v, [C_PATH + t // 6])
            val = plsc.load_gather(comb_v, [C_NODE_EMB + row * 6 + t % 6])
            plsc.store_scatter(out_v, [OFF_PATH + t],
                               plsc.bitcast(val, f32), mask=t < 90)

        for c in range(8):
            t = c * L + lanes
            val = comb_v[pl.ds(C_TAIL + c * L, L)].astype(f32)
            idx = jnp.where(t < 120, OFF_LINKS + t, OFF_SCAL + (t - 120))
            plsc.store_scatter(out_v, [idx], val, mask=t < 123)

        boss = plsc.load_gather(comb_v, [jnp.full((L,), BOSS_ID_POS, jnp.int32)])
        bval = plsc.load_gather(comb_v, [C_BOSS_EMB + boss * L + lanes])
        plsc.store_scatter(out_v, [OFF_BOSS + lanes], plsc.bitcast(bval, f32))

        gather.wait()
        acc0 = [None] * 4
        acc1 = [None] * 4
        for i in range(4):
            acc0[i] = cards_v[i, pl.ds(0, L)]
            acc1[i] = cards_v[i, pl.ds(L, L)]
        for i in range(4, N_DECK):
            k = i % 4
            acc0[k] = acc0[k] + cards_v[i, pl.ds(0, L)]
            acc1[k] = acc1[k] + cards_v[i, pl.ds(L, L)]
        scale = f32(1.0 / N_DECK)
        a0 = (acc0[0] + acc0[1]) + (acc0[2] + acc0[3])
        a1 = (acc1[0] + acc1[1]) + (acc1[2] + acc1[3])
        plsc.store_scatter(out_v, [OFF_CARD + lanes], a0 * scale)
        plsc.store_scatter(out_v, [OFF_CARD + L + lanes], a1 * scale)

        pltpu.sync_copy(out_v.at[pl.ds(960, OUT_PAD - 960)],
                        out_h.at[pl.ds(960, OUT_PAD - 960)])
        out1.wait()


_encode = pl.kernel(
    _body,
    out_type=jax.ShapeDtypeStruct((OUT_PAD,), jnp.float32),
    mesh=plsc.VectorSubcoreMesh(core_axis_name="c", subcore_axis_name="s",
                                num_cores=1, num_subcores=1),
    compiler_params=pltpu.CompilerParams(needs_layout_passes=False,
                                         use_tc_tiling_on_sc=False,
                                         skip_device_barrier=True),
    scratch_types=[
        pltpu.VMEM((C_TOTAL,), jnp.int32),
        pltpu.VMEM((DECK_PAD,), jnp.int32),
        pltpu.VMEM((DECK_PAD, 32), jnp.float32),
        pltpu.VMEM((OUT_PAD,), jnp.float32),
        pltpu.SemaphoreType.DMA,
        pltpu.SemaphoreType.DMA,
    ],
)


def kernel(collection, card_deck, potion, path_nodes, current_links,
           next_links, boss_id, current_health, max_health, current_floor,
           card_embed, potion_embed, node_embed, boss_embed):
    i32 = jnp.int32
    bits = lambda x: jax.lax.bitcast_convert_type(x.reshape(-1), i32)
    z = jnp.zeros((), i32)
    comb = jnp.concatenate([
        collection.astype(i32), jnp.zeros((7,), i32),
        potion.astype(i32), jnp.zeros((11,), i32),
        path_nodes.astype(i32), jnp.zeros((1,), i32),
        current_links.astype(i32).reshape(-1),
        next_links.astype(i32).reshape(-1),
        jnp.stack([jnp.asarray(current_health, i32),
                   jnp.asarray(max_health, i32),
                   jnp.asarray(current_floor, i32),
                   jnp.asarray(boss_id, i32)]),
        jnp.zeros((4,), i32),
        bits(potion_embed), jnp.zeros((8,), i32),
        bits(node_embed), jnp.zeros((4,), i32),
        bits(boss_embed),
    ])
    deck_p = jnp.pad(card_deck.astype(i32), (0, DECK_PAD - N_DECK))
    out = _encode(comb, deck_p, card_embed)
    return out[:OUT_LEN]


# speedup vs baseline: 1.1055x; 1.0237x over previous
import jax
import jax.numpy as jnp
from jax import lax
from jax.experimental import pallas as pl
from jax.experimental.pallas import tpu as pltpu
from jax.experimental.pallas import tpu_sc as plsc

L = 16

OFF_CARD = 1001
OFF_POTION = 1033
OFF_PATH = 1073
OFF_LINKS = 1163
OFF_BOSS = 1283
OFF_SCAL = 1299
OUT_LEN = 1302
OUT_PAD = 1312

N_DECK = 100
DECK_PAD = 112

C_DECK = 0
C_COLL = 112
C_POTION = 1120
C_PATH = 1136
C_TAIL = 1152
C_POT_EMB = 1280
C_NODE_EMB = 1696
C_BOSS_EMB = 1760
C_TOTAL = 2080
BOSS_ID_POS = C_TAIL + 123


def _body(comb_h, card_emb_h, out_h,
          comb_v, cards_v, out_v, sem, gsem):
    @pl.when(jnp.logical_and(lax.axis_index("c") == 0, lax.axis_index("s") == 0))
    def _():
        pltpu.sync_copy(comb_h, comb_v)
        gather = pltpu.async_copy(card_emb_h.at[comb_v.at[pl.ds(C_DECK, DECK_PAD)]],
                                  cards_v, gsem)

        lanes = lax.iota(jnp.int32, L)
        f32 = jnp.float32

        for i in range(62):
            out_v[pl.ds(i * L, L)] = comb_v[pl.ds(C_COLL + i * L, L)].astype(f32)
        t = 62 * L + lanes
        plsc.store_scatter(out_v, [t],
                           comb_v[pl.ds(C_COLL + 62 * L, L)].astype(f32),
                           mask=t < OFF_CARD)

        out1 = pltpu.async_copy(out_v.at[pl.ds(0, 960)],
                                out_h.at[pl.ds(0, 960)], sem)

        for c in range(3):
            t = c * L + lanes
            row = plsc.load_gather(comb_v, [C_POTION + t // 8])
            val = plsc.load_gather(comb_v, [C_POT_EMB + row * 8 + t % 8])
            plsc.store_scatter(out_v, [OFF_POTION + t],
                               plsc.bitcast(val, f32), mask=t < 40)

        for c in range(6):
            t = c * L + lanes
            row = plsc.load_gather(comb_v, [C_PATH + t // 6])
            val = plsc.load_gather(comb_v, [C_NODE_EMB + row * 6 + t % 6])
            plsc.store_scatter(out_v, [OFF_PATH + t],
                               plsc.bitcast(val, f32), mask=t < 90)

        for c in range(8):
            t = c * L + lanes
            val = comb_v[pl.ds(C_TAIL + c * L, L)].astype(f32)
            idx = jnp.where(t < 120, OFF_LINKS + t, OFF_SCAL + (t - 120))
            plsc.store_scatter(out_v, [idx], val, mask=t < 123)

        boss = plsc.load_gather(comb_v, [jnp.full((L,), BOSS_ID_POS, jnp.int32)])
        bval = plsc.load_gather(comb_v, [C_BOSS_EMB + boss * L + lanes])
        plsc.store_scatter(out_v, [OFF_BOSS + lanes], plsc.bitcast(bval, f32))

        gather.wait()

        def acc(i, carry):
            a0, a1 = carry
            return (a0 + cards_v[i, pl.ds(0, L)], a1 + cards_v[i, pl.ds(L, L)])

        zero = jnp.zeros((L,), f32)
        a0, a1 = lax.fori_loop(0, N_DECK, acc, (zero, zero))
        scale = f32(1.0 / N_DECK)
        plsc.store_scatter(out_v, [OFF_CARD + lanes], a0 * scale)
        plsc.store_scatter(out_v, [OFF_CARD + L + lanes], a1 * scale)

        pltpu.sync_copy(out_v.at[pl.ds(960, OUT_PAD - 960)],
                        out_h.at[pl.ds(960, OUT_PAD - 960)])
        out1.wait()


_encode = pl.kernel(
    _body,
    out_type=jax.ShapeDtypeStruct((OUT_PAD,), jnp.float32),
    mesh=plsc.VectorSubcoreMesh(core_axis_name="c", subcore_axis_name="s",
                                num_cores=1, num_subcores=1),
    compiler_params=pltpu.CompilerParams(needs_layout_passes=False,
                                         use_tc_tiling_on_sc=False,
                                         skip_device_barrier=True),
    scratch_types=[
        pltpu.VMEM((C_TOTAL,), jnp.int32),
        pltpu.VMEM((DECK_PAD, 32), jnp.float32),
        pltpu.VMEM((OUT_PAD,), jnp.float32),
        pltpu.SemaphoreType.DMA,
        pltpu.SemaphoreType.DMA,
    ],
)


def kernel(collection, card_deck, potion, path_nodes, current_links,
           next_links, boss_id, current_health, max_health, current_floor,
           card_embed, potion_embed, node_embed, boss_embed):
    i32 = jnp.int32
    bits = lambda x: jax.lax.bitcast_convert_type(x.reshape(-1), i32)
    z = jnp.zeros((), i32)
    comb = jnp.concatenate([
        card_deck.astype(i32), jnp.zeros((12,), i32),
        collection.astype(i32), jnp.zeros((7,), i32),
        potion.astype(i32), jnp.zeros((11,), i32),
        path_nodes.astype(i32), jnp.zeros((1,), i32),
        current_links.astype(i32).reshape(-1),
        next_links.astype(i32).reshape(-1),
        jnp.stack([jnp.asarray(current_health, i32),
                   jnp.asarray(max_health, i32),
                   jnp.asarray(current_floor, i32),
                   jnp.asarray(boss_id, i32)]),
        jnp.zeros((4,), i32),
        bits(potion_embed), jnp.zeros((8,), i32),
        bits(node_embed), jnp.zeros((4,), i32),
        bits(boss_embed),
    ])
    out = _encode(comb, card_embed)
    return out[:OUT_LEN]
